# baseline (speedup 1.0000x reference)
import functools

import jax
import jax.numpy as jnp
from jax import lax
from jax.experimental import pallas as pl
from jax.experimental.pallas import tpu as pltpu
from jax.experimental.pallas import tpu_sc as plsc

_SC_PARAMS = None


def _sc_mesh():
    info = plsc.get_sparse_core_info()
    nc, ns = info.num_cores, info.num_subcores
    mesh = plsc.VectorSubcoreMesh(core_axis_name="c", subcore_axis_name="s")
    return mesh, nc, ns


def _sc_compiler_params():
    return pltpu.CompilerParams(needs_layout_passes=False)


@functools.lru_cache(maxsize=None)
def _make_sc_deg(N, E):
    mesh, nc, ns = _sc_mesh()
    nw = nc * ns
    assert E % (nw * 16) == 0
    ep = E // nw

    @functools.partial(
        pl.kernel, mesh=mesh,
        compiler_params=_sc_compiler_params(),
        out_type=jax.ShapeDtypeStruct((nw, N), jnp.float32),
        scratch_types=[
            pltpu.VMEM((ep,), jnp.int32),
            pltpu.VMEM((ep,), jnp.float32),
            pltpu.VMEM((N,), jnp.float32),
        ],
    )
    def deg_kernel(src_hbm, ew_hbm, out_hbm, src_v, ew_v, acc_v):
        wid = lax.axis_index("s") * nc + lax.axis_index("c")
        base = wid * ep

        @plsc.parallel_loop(0, N // 16, unroll=8)
        def _zero(i):
            acc_v[pl.ds(i * 16, 16)] = jnp.zeros((16,), jnp.float32)

        pltpu.sync_copy(src_hbm.at[pl.ds(base, ep)], src_v)
        pltpu.sync_copy(ew_hbm.at[pl.ds(base, ep)], ew_v)

        @plsc.parallel_loop(0, ep // 16, unroll=8)
        def _body(g):
            idx = src_v[pl.ds(g * 16, 16)]
            w = ew_v[pl.ds(g * 16, 16)]
            plsc.addupdate_scatter(acc_v, [idx], w)

        pltpu.sync_copy(acc_v, out_hbm.at[wid])

    return deg_kernel


@functools.lru_cache(maxsize=None)
def _make_sc_spmm(N, E, D, CH):
    mesh, nc, ns = _sc_mesh()
    nw = nc * ns
    assert D % nw == 0 and E % CH == 0 and CH % 16 == 0
    F = D // nw
    nch = E // CH

    assert nch % 2 == 0

    @functools.partial(
        pl.kernel, mesh=mesh,
        compiler_params=_sc_compiler_params(),
        out_type=jax.ShapeDtypeStruct((D, N), jnp.float32),
        scratch_types=(
            [pltpu.VMEM((N,), jnp.float32) for _ in range(F)]
            + [pltpu.VMEM((N,), jnp.float32) for _ in range(F)]
            + [
                pltpu.VMEM((CH,), jnp.int32),
                pltpu.VMEM((CH,), jnp.int32),
                pltpu.VMEM((CH,), jnp.float32),
                pltpu.VMEM((CH,), jnp.float32),
                pltpu.SemaphoreType.DMA,
                pltpu.SemaphoreType.DMA,
            ]
        ),
    )
    def spmm_kernel(pk_hbm, ew_hbm, featT_hbm, out_hbm, *scratch):
        feat_vs = scratch[0:F]
        acc_vs = scratch[F:2 * F]
        pk_v0, pk_v1, ew_v0, ew_v1, sem0, sem1 = scratch[2 * F:]
        wid = lax.axis_index("s") * nc + lax.axis_index("c")
        f0 = wid * F
        sems = (sem0, sem1)
        pk_b = (pk_v0, pk_v1)
        ew_b = (ew_v0, ew_v1)

        def start(c, b):
            base = c * CH
            pltpu.async_copy(pk_hbm.at[pl.ds(base, CH)], pk_b[b], sems[b])
            pltpu.async_copy(ew_hbm.at[pl.ds(base, CH)], ew_b[b], sems[b])

        def wait(b):
            pltpu.make_async_copy(pk_hbm.at[pl.ds(0, CH)], pk_b[b], sems[b]).wait()
            pltpu.make_async_copy(ew_hbm.at[pl.ds(0, CH)], ew_b[b], sems[b]).wait()

        start(0, 0)
        for f in range(F):
            pltpu.sync_copy(featT_hbm.at[f0 + f], feat_vs[f])

        @plsc.parallel_loop(0, N // 16, unroll=8)
        def _zero(i):
            z = jnp.zeros((16,), jnp.float32)
            for f in range(F):
                acc_vs[f][pl.ds(i * 16, 16)] = z

        def outer(i, _):
            for b in range(2):
                c = i * 2 + b

                @pl.when(c + 1 < nch)
                def _():
                    start(c + 1, 1 - b)

                wait(b)

                @plsc.parallel_loop(0, CH // 16, unroll=16)
                def _body(g):
                    pk16 = pk_b[b][pl.ds(g * 16, 16)]
                    w16 = ew_b[b][pl.ds(g * 16, 16)]
                    s16 = jnp.bitwise_and(pk16, 16383)
                    d16 = lax.shift_right_logical(pk16, 14)
                    for f in range(F):
                        v = plsc.load_gather(feat_vs[f], [s16])
                        plsc.addupdate_scatter(acc_vs[f], [d16], v * w16)
            return 0
        lax.fori_loop(0, nch // 2, outer, 0)

        for f in range(F):
            pltpu.sync_copy(acc_vs[f], out_hbm.at[f0 + f])

    return spmm_kernel


def _dot(a, b):
    return lax.dot_general(a, b, (((1,), (0,)), ((), ())),
                           precision=lax.Precision.HIGHEST,
                           preferred_element_type=jnp.float32)


def _dotT(aT, b):
    return lax.dot_general(aT, b, (((0,), (0,)), ((), ())),
                           precision=lax.Precision.HIGHEST,
                           preferred_element_type=jnp.float32)


def _sigmoid(t):
    return 1.0 / (1.0 + jnp.exp(-t))


@functools.lru_cache(maxsize=None)
def _make_tc_pre(N, D, NW, BN):
    grid = (N // BN,)

    def body(degT, x, h, wxz, whz, wxr, whr, wxh, bz, br, bh,
             dis_o, gz_o, gr_o, gxh_o, xs_o, hs_o):
        deg = jnp.sum(degT[...], axis=1)
        dis = jnp.where(deg > 0, lax.rsqrt(jnp.where(deg > 0, deg, 1.0)), 0.0)
        d = dis[:, None]
        dis_o[...] = d
        xx = x[...]
        hh = h[...]
        xs_o[...] = d * xx
        hs_o[...] = d * hh
        gz_o[...] = _dot(xx, wxz[...]) + _dot(hh, whz[...]) + bz[...]
        gr_o[...] = _dot(xx, wxr[...]) + _dot(hh, whr[...]) + br[...]
        gxh_o[...] = _dot(xx, wxh[...]) + bh[...]

    row_blk = pl.BlockSpec((BN, D), lambda i: (i, 0))
    w_blk = pl.BlockSpec((D, D), lambda i: (0, 0))
    b_blk = pl.BlockSpec((1, D), lambda i: (0, 0))
    return pl.pallas_call(
        body, grid=grid,
        in_specs=[pl.BlockSpec((BN, NW), lambda i: (i, 0)), row_blk, row_blk,
                  w_blk, w_blk, w_blk, w_blk, w_blk, b_blk, b_blk, b_blk],
        out_specs=[pl.BlockSpec((BN, 1), lambda i: (i, 0)),
                   row_blk, row_blk, row_blk, row_blk, row_blk],
        out_shape=[jax.ShapeDtypeStruct((N, 1), jnp.float32),
                   jax.ShapeDtypeStruct((N, D), jnp.float32),
                   jax.ShapeDtypeStruct((N, D), jnp.float32),
                   jax.ShapeDtypeStruct((N, D), jnp.float32),
                   jax.ShapeDtypeStruct((N, D), jnp.float32),
                   jax.ShapeDtypeStruct((N, D), jnp.float32)],
    )


@functools.lru_cache(maxsize=None)
def _make_tc_mid(N, D, BN):
    grid = (N // BN,)

    def body(gz, gr, gxh, txt, tht, dis, h, wxz1, whz1, wxr1, whr1, whh0,
             z_o, hrs_o, gh_o):
        d = dis[...]
        tx = txt[...]
        th = tht[...]
        z = _sigmoid(gz[...] - d * (_dotT(tx, wxz1[...]) + _dotT(th, whz1[...])))
        r = _sigmoid(gr[...] - d * (_dotT(tx, wxr1[...]) + _dotT(th, whr1[...])))
        hr = h[...] * r
        z_o[...] = z
        hrs_o[...] = d * hr
        gh_o[...] = gxh[...] + _dot(hr, whh0[...])

    row_blk = pl.BlockSpec((BN, D), lambda i: (i, 0))
    t_blk = pl.BlockSpec((D, BN), lambda i: (0, i))
    w_blk = pl.BlockSpec((D, D), lambda i: (0, 0))
    return pl.pallas_call(
        body, grid=grid,
        in_specs=[row_blk, row_blk, row_blk, t_blk, t_blk,
                  pl.BlockSpec((BN, 1), lambda i: (i, 0)), row_blk,
                  w_blk, w_blk, w_blk, w_blk, w_blk],
        out_specs=[row_blk, row_blk, row_blk],
        out_shape=[jax.ShapeDtypeStruct((N, D), jnp.float32),
                   jax.ShapeDtypeStruct((N, D), jnp.float32),
                   jax.ShapeDtypeStruct((N, D), jnp.float32)],
    )


@functools.lru_cache(maxsize=None)
def _make_tc_fin(N, D, BN):
    grid = (N // BN,)

    def body(gh, txt, thrt, dis, z, h, wxh1, whh1, wlin, blin, out_o, h_o):
        d = dis[...]
        ht = jnp.tanh(gh[...] - d * (_dotT(txt[...], wxh1[...]) +
                                     _dotT(thrt[...], whh1[...])))
        zz = z[...]
        hv = zz * h[...] + (1.0 - zz) * ht
        h_o[...] = hv
        v = _dot(jnp.maximum(hv, 0.0), wlin[...]) + blin[...]
        out_o[...] = jnp.maximum(v, 0.0) + jnp.log1p(jnp.exp(-jnp.abs(v)))

    row_blk = pl.BlockSpec((BN, D), lambda i: (i, 0))
    t_blk = pl.BlockSpec((D, BN), lambda i: (0, i))
    w_blk = pl.BlockSpec((D, D), lambda i: (0, 0))
    return pl.pallas_call(
        body, grid=grid,
        in_specs=[row_blk, t_blk, t_blk,
                  pl.BlockSpec((BN, 1), lambda i: (i, 0)), row_blk, row_blk,
                  w_blk, w_blk, pl.BlockSpec((D, 1), lambda i: (0, 0)),
                  pl.BlockSpec((1, 1), lambda i: (0, 0))],
        out_specs=[pl.BlockSpec((BN, 1), lambda i: (i, 0)), row_blk],
        out_shape=[jax.ShapeDtypeStruct((N, 1), jnp.float32),
                   jax.ShapeDtypeStruct((N, D), jnp.float32)],
    )


def kernel(x, edge_index, edge_weight, h,
           W_xz, b_xz, W_hz, b_hz, W_xr, b_xr, W_hr, b_hr,
           W_xh, b_xh, W_hh, b_hh, W_lin, b_lin):
    N, D = x.shape
    E = edge_index.shape[1]
    info = plsc.get_sparse_core_info()
    NW = info.num_cores * info.num_subcores
    BN = 2048
    CH = 6400
    NP = -(-N // BN) * BN

    assert N <= 16384

    src = edge_index[0]
    dst = edge_index[1]
    pk = src + (dst << 14)
    xp = jnp.pad(x, ((0, NP - N), (0, 0)))
    hp = jnp.pad(h, ((0, NP - N), (0, 0)))

    deg_parts = _make_sc_deg(NP, E)(src, edge_weight)

    bz = (b_xz + b_hz).reshape(1, D)
    br = (b_xr + b_hr).reshape(1, D)
    bh = (b_xh + b_hh).reshape(1, D)
    dis, Gz, Gr, Gxh, xs, hs = _make_tc_pre(NP, D, NW, BN)(
        deg_parts.T, xp, hp, W_xz[0], W_hz[0], W_xr[0], W_hr[0], W_xh[0],
        bz, br, bh)

    spmm = _make_sc_spmm(NP, E, D, CH)
    TxT = spmm(pk, edge_weight, xs.T)
    ThT = spmm(pk, edge_weight, hs.T)

    Z, hrs, Gh = _make_tc_mid(NP, D, BN)(
        Gz, Gr, Gxh, TxT, ThT, dis, hp,
        W_xz[1], W_hz[1], W_xr[1], W_hr[1], W_hh[0])

    ThrT = spmm(pk, edge_weight, hrs.T)

    out, H = _make_tc_fin(NP, D, BN)(
        Gh, TxT, ThrT, dis, Z, hp, W_xh[1], W_hh[1], W_lin,
        b_lin.reshape(1, 1))
    return (out[:N], H[:N])


# speedup vs baseline: 1.1190x; 1.1190x over previous
import functools

import jax
import jax.numpy as jnp
from jax import lax
from jax.experimental import pallas as pl
from jax.experimental.pallas import tpu as pltpu
from jax.experimental.pallas import tpu_sc as plsc

_SC_PARAMS = None


def _sc_mesh():
    info = plsc.get_sparse_core_info()
    nc, ns = info.num_cores, info.num_subcores
    mesh = plsc.VectorSubcoreMesh(core_axis_name="c", subcore_axis_name="s")
    return mesh, nc, ns


def _sc_compiler_params():
    return pltpu.CompilerParams(needs_layout_passes=False)


@functools.lru_cache(maxsize=None)
def _make_sc_deg(N, E):
    mesh, nc, ns = _sc_mesh()
    nw = nc * ns
    assert E % (nw * 16) == 0
    ep = E // nw

    @functools.partial(
        pl.kernel, mesh=mesh,
        compiler_params=_sc_compiler_params(),
        out_type=jax.ShapeDtypeStruct((nw, N), jnp.float32),
        scratch_types=[
            pltpu.VMEM((ep,), jnp.int32),
            pltpu.VMEM((ep,), jnp.float32),
            pltpu.VMEM((N,), jnp.float32),
        ],
    )
    def deg_kernel(src_hbm, ew_hbm, out_hbm, src_v, ew_v, acc_v):
        wid = lax.axis_index("s") * nc + lax.axis_index("c")
        base = wid * ep

        @plsc.parallel_loop(0, N // 16, unroll=8)
        def _zero(i):
            acc_v[pl.ds(i * 16, 16)] = jnp.zeros((16,), jnp.float32)

        pltpu.sync_copy(src_hbm.at[pl.ds(base, ep)], src_v)
        pltpu.sync_copy(ew_hbm.at[pl.ds(base, ep)], ew_v)

        @plsc.parallel_loop(0, ep // 16, unroll=8)
        def _body(g):
            idx = src_v[pl.ds(g * 16, 16)]
            w = ew_v[pl.ds(g * 16, 16)]
            plsc.addupdate_scatter(acc_v, [idx], w)

        pltpu.sync_copy(acc_v, out_hbm.at[wid])

    return deg_kernel


@functools.lru_cache(maxsize=None)
def _make_sc_spmm(N, E, D, CH):
    mesh, nc, ns = _sc_mesh()
    nw = nc * ns
    DP = D // 2
    assert DP % nw == 0 and E % CH == 0 and CH % 16 == 0
    FP = DP // nw
    F = 2 * FP
    nch = E // CH
    assert nch % 2 == 0

    @functools.partial(
        pl.kernel, mesh=mesh,
        compiler_params=_sc_compiler_params(),
        out_type=jax.ShapeDtypeStruct((D, N), jnp.float32),
        scratch_types=[
            pltpu.VMEM((FP, N), jnp.int32),
            pltpu.VMEM((F, N), jnp.float32),
            pltpu.VMEM((CH,), jnp.int32),
            pltpu.VMEM((CH,), jnp.int32),
            pltpu.VMEM((CH,), jnp.float32),
            pltpu.VMEM((CH,), jnp.float32),
            pltpu.SemaphoreType.DMA,
            pltpu.SemaphoreType.DMA,
        ],
    )
    def spmm_kernel(pk_hbm, ew_hbm, featP_hbm, out_hbm,
                    feat_v, acc_v, pk_v0, pk_v1, ew_v0, ew_v1, sem0, sem1):
        wid = lax.axis_index("s") * nc + lax.axis_index("c")
        p0 = wid * FP
        sems = (sem0, sem1)
        pk_b = (pk_v0, pk_v1)
        ew_b = (ew_v0, ew_v1)

        def start(c, b):
            base = c * CH
            pltpu.async_copy(pk_hbm.at[pl.ds(base, CH)], pk_b[b], sems[b])
            pltpu.async_copy(ew_hbm.at[pl.ds(base, CH)], ew_b[b], sems[b])

        def wait(b):
            pltpu.make_async_copy(pk_hbm.at[pl.ds(0, CH)], pk_b[b], sems[b]).wait()
            pltpu.make_async_copy(ew_hbm.at[pl.ds(0, CH)], ew_b[b], sems[b]).wait()

        start(0, 0)
        pltpu.sync_copy(featP_hbm.at[pl.ds(p0, FP)], feat_v)

        @plsc.parallel_loop(0, N // 16, unroll=8)
        def _zero(i):
            z = jnp.zeros((16,), jnp.float32)
            for f in range(F):
                acc_v[f, pl.ds(i * 16, 16)] = z

        def outer(i, _):
            for b in range(2):
                c = i * 2 + b

                @pl.when(c + 1 < nch)
                def _():
                    start(c + 1, 1 - b)

                wait(b)

                @plsc.parallel_loop(0, CH // 16, unroll=16)
                def _body(g):
                    pk16 = pk_b[b][pl.ds(g * 16, 16)]
                    w16 = ew_b[b][pl.ds(g * 16, 16)]
                    s16 = jnp.bitwise_and(pk16, 16383)
                    d16 = lax.shift_right_logical(pk16, 14)
                    for j in range(FP):
                        jidx = jnp.full((16,), j, jnp.int32)
                        vp = plsc.load_gather(feat_v, [jidx, s16])
                        lo = plsc.bitcast(lax.shift_left(vp, 16), jnp.float32)
                        hi = plsc.bitcast(
                            jnp.bitwise_and(vp, jnp.int32(-65536)), jnp.float32)
                        lidx = jnp.full((16,), 2 * j, jnp.int32)
                        hidx = jnp.full((16,), 2 * j + 1, jnp.int32)
                        plsc.addupdate_scatter(acc_v, [lidx, d16], lo * w16)
                        plsc.addupdate_scatter(acc_v, [hidx, d16], hi * w16)
            return 0
        lax.fori_loop(0, nch // 2, outer, 0)

        pltpu.sync_copy(acc_v, out_hbm.at[pl.ds(2 * p0, F)])

    return spmm_kernel


def _dot(a, b):
    return lax.dot_general(a, b, (((1,), (0,)), ((), ())),
                           precision=lax.Precision.HIGHEST,
                           preferred_element_type=jnp.float32)


def _dotT(aT, b):
    return lax.dot_general(aT, b, (((0,), (0,)), ((), ())),
                           precision=lax.Precision.HIGHEST,
                           preferred_element_type=jnp.float32)


def _sigmoid(t):
    return 1.0 / (1.0 + jnp.exp(-t))


def _pack_pairs(a):
    hw = a.shape[1] // 2
    lo = lax.bitcast_convert_type(a[:, :hw].astype(jnp.bfloat16), jnp.uint16)
    hi = lax.bitcast_convert_type(a[:, hw:].astype(jnp.bfloat16), jnp.uint16)
    return (lo.astype(jnp.int32) | (hi.astype(jnp.int32) << 16))


@functools.lru_cache(maxsize=None)
def _make_tc_pre(N, D, NW, BN):
    grid = (N // BN,)

    def body(degT, x, h, wxz, whz, wxr, whr, wxh, bz, br, bh,
             dis_o, gz_o, gr_o, gxh_o, xs_o, hs_o):
        deg = jnp.sum(degT[...], axis=1)
        dis = jnp.where(deg > 0, lax.rsqrt(jnp.where(deg > 0, deg, 1.0)), 0.0)
        d = dis[:, None]
        dis_o[...] = d
        xx = x[...]
        hh = h[...]
        xs_o[...] = _pack_pairs(d * xx)
        hs_o[...] = _pack_pairs(d * hh)
        gz_o[...] = _dot(xx, wxz[...]) + _dot(hh, whz[...]) + bz[...]
        gr_o[...] = _dot(xx, wxr[...]) + _dot(hh, whr[...]) + br[...]
        gxh_o[...] = _dot(xx, wxh[...]) + bh[...]

    row_blk = pl.BlockSpec((BN, D), lambda i: (i, 0))
    pk_blk = pl.BlockSpec((BN, D // 2), lambda i: (i, 0))
    w_blk = pl.BlockSpec((D, D), lambda i: (0, 0))
    b_blk = pl.BlockSpec((1, D), lambda i: (0, 0))
    return pl.pallas_call(
        body, grid=grid,
        in_specs=[pl.BlockSpec((BN, NW), lambda i: (i, 0)), row_blk, row_blk,
                  w_blk, w_blk, w_blk, w_blk, w_blk, b_blk, b_blk, b_blk],
        out_specs=[pl.BlockSpec((BN, 1), lambda i: (i, 0)),
                   row_blk, row_blk, row_blk, pk_blk, pk_blk],
        out_shape=[jax.ShapeDtypeStruct((N, 1), jnp.float32),
                   jax.ShapeDtypeStruct((N, D), jnp.float32),
                   jax.ShapeDtypeStruct((N, D), jnp.float32),
                   jax.ShapeDtypeStruct((N, D), jnp.float32),
                   jax.ShapeDtypeStruct((N, D // 2), jnp.int32),
                   jax.ShapeDtypeStruct((N, D // 2), jnp.int32)],
    )


@functools.lru_cache(maxsize=None)
def _make_tc_mid(N, D, BN):
    grid = (N // BN,)

    def body(gz, gr, gxh, txt, tht, dis, h, wxz1, whz1, wxr1, whr1, whh0,
             z_o, hrs_o, gh_o):
        d = dis[...]
        tx = txt[...]
        th = tht[...]
        z = _sigmoid(gz[...] - d * (_dotT(tx, wxz1[...]) + _dotT(th, whz1[...])))
        r = _sigmoid(gr[...] - d * (_dotT(tx, wxr1[...]) + _dotT(th, whr1[...])))
        hr = h[...] * r
        z_o[...] = z
        hrs_o[...] = _pack_pairs(d * hr)
        gh_o[...] = gxh[...] + _dot(hr, whh0[...])

    row_blk = pl.BlockSpec((BN, D), lambda i: (i, 0))
    pk_blk = pl.BlockSpec((BN, D // 2), lambda i: (i, 0))
    t_blk = pl.BlockSpec((D, BN), lambda i: (0, i))
    w_blk = pl.BlockSpec((D, D), lambda i: (0, 0))
    return pl.pallas_call(
        body, grid=grid,
        in_specs=[row_blk, row_blk, row_blk, t_blk, t_blk,
                  pl.BlockSpec((BN, 1), lambda i: (i, 0)), row_blk,
                  w_blk, w_blk, w_blk, w_blk, w_blk],
        out_specs=[row_blk, pk_blk, row_blk],
        out_shape=[jax.ShapeDtypeStruct((N, D), jnp.float32),
                   jax.ShapeDtypeStruct((N, D // 2), jnp.int32),
                   jax.ShapeDtypeStruct((N, D), jnp.float32)],
    )


@functools.lru_cache(maxsize=None)
def _make_tc_fin(N, D, BN):
    grid = (N // BN,)

    def body(gh, txt, thrt, dis, z, h, wxh1, whh1, wlin, blin, out_o, h_o):
        d = dis[...]
        ht = jnp.tanh(gh[...] - d * (_dotT(txt[...], wxh1[...]) +
                                     _dotT(thrt[...], whh1[...])))
        zz = z[...]
        hv = zz * h[...] + (1.0 - zz) * ht
        h_o[...] = hv
        v = _dot(jnp.maximum(hv, 0.0), wlin[...]) + blin[...]
        out_o[...] = jnp.maximum(v, 0.0) + jnp.log1p(jnp.exp(-jnp.abs(v)))

    row_blk = pl.BlockSpec((BN, D), lambda i: (i, 0))
    t_blk = pl.BlockSpec((D, BN), lambda i: (0, i))
    w_blk = pl.BlockSpec((D, D), lambda i: (0, 0))
    return pl.pallas_call(
        body, grid=grid,
        in_specs=[row_blk, t_blk, t_blk,
                  pl.BlockSpec((BN, 1), lambda i: (i, 0)), row_blk, row_blk,
                  w_blk, w_blk, pl.BlockSpec((D, 1), lambda i: (0, 0)),
                  pl.BlockSpec((1, 1), lambda i: (0, 0))],
        out_specs=[pl.BlockSpec((BN, 1), lambda i: (i, 0)), row_blk],
        out_shape=[jax.ShapeDtypeStruct((N, 1), jnp.float32),
                   jax.ShapeDtypeStruct((N, D), jnp.float32)],
    )


def kernel(x, edge_index, edge_weight, h,
           W_xz, b_xz, W_hz, b_hz, W_xr, b_xr, W_hr, b_hr,
           W_xh, b_xh, W_hh, b_hh, W_lin, b_lin):
    N, D = x.shape
    E = edge_index.shape[1]
    info = plsc.get_sparse_core_info()
    NW = info.num_cores * info.num_subcores
    BN = 2048
    CH = 6400
    NP = -(-N // BN) * BN

    assert N <= 16384

    src = edge_index[0]
    dst = edge_index[1]
    pk = src + (dst << 14)
    xp = jnp.pad(x, ((0, NP - N), (0, 0)))
    hp = jnp.pad(h, ((0, NP - N), (0, 0)))

    deg_parts = _make_sc_deg(NP, E)(src, edge_weight)

    bz = (b_xz + b_hz).reshape(1, D)
    br = (b_xr + b_hr).reshape(1, D)
    bh = (b_xh + b_hh).reshape(1, D)
    dis, Gz, Gr, Gxh, xs_pk, hs_pk = _make_tc_pre(NP, D, NW, BN)(
        deg_parts.T, xp, hp, W_xz[0], W_hz[0], W_xr[0], W_hr[0], W_xh[0],
        bz, br, bh)

    DP = D // 2
    perm = jnp.array([p + b * DP for p in range(DP) for b in (0, 1)],
                     dtype=jnp.int32)

    spmm = _make_sc_spmm(NP, E, D, CH)
    TxT = spmm(pk, edge_weight, xs_pk.T)
    ThT = spmm(pk, edge_weight, hs_pk.T)

    Z, hrs_pk, Gh = _make_tc_mid(NP, D, BN)(
        Gz, Gr, Gxh, TxT, ThT, dis, hp,
        W_xz[1][perm], W_hz[1][perm], W_xr[1][perm], W_hr[1][perm], W_hh[0])

    ThrT = spmm(pk, edge_weight, hrs_pk.T)

    out, H = _make_tc_fin(NP, D, BN)(
        Gh, TxT, ThrT, dis, Z, hp, W_xh[1][perm], W_hh[1][perm], W_lin,
        b_lin.reshape(1, 1))
    return (out[:N], H[:N])


# speedup vs baseline: 1.1349x; 1.0142x over previous
import functools

import jax
import jax.numpy as jnp
from jax import lax
from jax.experimental import pallas as pl
from jax.experimental.pallas import tpu as pltpu
from jax.experimental.pallas import tpu_sc as plsc

_SC_PARAMS = None


def _sc_mesh():
    info = plsc.get_sparse_core_info()
    nc, ns = info.num_cores, info.num_subcores
    mesh = plsc.VectorSubcoreMesh(core_axis_name="c", subcore_axis_name="s")
    return mesh, nc, ns


def _sc_compiler_params():
    return pltpu.CompilerParams(needs_layout_passes=False)


@functools.lru_cache(maxsize=None)
def _make_sc_deg(N, E):
    mesh, nc, ns = _sc_mesh()
    nw = nc * ns
    assert E % (nw * 16) == 0
    ep = E // nw

    @functools.partial(
        pl.kernel, mesh=mesh,
        compiler_params=_sc_compiler_params(),
        out_type=(jax.ShapeDtypeStruct((nw, N), jnp.float32),
                  jax.ShapeDtypeStruct((E,), jnp.int32)),
        scratch_types=[
            pltpu.VMEM((ep,), jnp.int32),
            pltpu.VMEM((ep,), jnp.int32),
            pltpu.VMEM((ep,), jnp.float32),
            pltpu.VMEM((N,), jnp.float32),
        ],
    )
    def deg_kernel(src_hbm, dst_hbm, ew_hbm, out_hbm, pk_hbm,
                   src_v, dst_v, ew_v, acc_v):
        wid = lax.axis_index("s") * nc + lax.axis_index("c")
        base = wid * ep

        @plsc.parallel_loop(0, N // 16, unroll=8)
        def _zero(i):
            acc_v[pl.ds(i * 16, 16)] = jnp.zeros((16,), jnp.float32)

        pltpu.sync_copy(src_hbm.at[pl.ds(base, ep)], src_v)
        pltpu.sync_copy(dst_hbm.at[pl.ds(base, ep)], dst_v)
        pltpu.sync_copy(ew_hbm.at[pl.ds(base, ep)], ew_v)

        @plsc.parallel_loop(0, ep // 16, unroll=8)
        def _body(g):
            sl = pl.ds(g * 16, 16)
            idx = src_v[sl]
            w = ew_v[sl]
            plsc.addupdate_scatter(acc_v, [idx], w)
            dst_v[sl] = idx + lax.shift_left(dst_v[sl], 14)

        pltpu.sync_copy(acc_v, out_hbm.at[wid])
        pltpu.sync_copy(dst_v, pk_hbm.at[pl.ds(base, ep)])

    return deg_kernel


@functools.lru_cache(maxsize=None)
def _make_sc_spmm(N, E, D, CH):
    mesh, nc, ns = _sc_mesh()
    nw = nc * ns
    DP = D // 2
    assert DP % nw == 0 and E % CH == 0 and CH % 16 == 0
    FP = DP // nw
    F = 2 * FP
    nch = E // CH
    assert nch % 2 == 0

    @functools.partial(
        pl.kernel, mesh=mesh,
        compiler_params=_sc_compiler_params(),
        out_type=jax.ShapeDtypeStruct((D, N), jnp.float32),
        scratch_types=[
            pltpu.VMEM((FP, N), jnp.int32),
            pltpu.VMEM((F, N), jnp.float32),
            pltpu.VMEM((CH,), jnp.int32),
            pltpu.VMEM((CH,), jnp.int32),
            pltpu.VMEM((CH,), jnp.float32),
            pltpu.VMEM((CH,), jnp.float32),
            pltpu.SemaphoreType.DMA,
            pltpu.SemaphoreType.DMA,
        ],
    )
    def spmm_kernel(pk_hbm, ew_hbm, featP_hbm, out_hbm,
                    feat_v, acc_v, pk_v0, pk_v1, ew_v0, ew_v1, sem0, sem1):
        wid = lax.axis_index("s") * nc + lax.axis_index("c")
        p0 = wid * FP
        sems = (sem0, sem1)
        pk_b = (pk_v0, pk_v1)
        ew_b = (ew_v0, ew_v1)

        def start(c, b):
            base = c * CH
            pltpu.async_copy(pk_hbm.at[pl.ds(base, CH)], pk_b[b], sems[b])
            pltpu.async_copy(ew_hbm.at[pl.ds(base, CH)], ew_b[b], sems[b])

        def wait(b):
            pltpu.make_async_copy(pk_hbm.at[pl.ds(0, CH)], pk_b[b], sems[b]).wait()
            pltpu.make_async_copy(ew_hbm.at[pl.ds(0, CH)], ew_b[b], sems[b]).wait()

        start(0, 0)
        pltpu.sync_copy(featP_hbm.at[pl.ds(p0, FP)], feat_v)

        @plsc.parallel_loop(0, N // 16, unroll=8)
        def _zero(i):
            z = jnp.zeros((16,), jnp.float32)
            for f in range(F):
                acc_v[f, pl.ds(i * 16, 16)] = z

        def outer(i, _):
            for b in range(2):
                c = i * 2 + b

                @pl.when(c + 1 < nch)
                def _():
                    start(c + 1, 1 - b)

                wait(b)

                @plsc.parallel_loop(0, CH // 16, unroll=20)
                def _body(g):
                    pk16 = pk_b[b][pl.ds(g * 16, 16)]
                    w16 = ew_b[b][pl.ds(g * 16, 16)]
                    s16 = jnp.bitwise_and(pk16, 16383)
                    d16 = lax.shift_right_logical(pk16, 14)
                    for j in range(FP):
                        jidx = jnp.full((16,), j, jnp.int32)
                        vp = plsc.load_gather(feat_v, [jidx, s16])
                        lo = plsc.bitcast(lax.shift_left(vp, 16), jnp.float32)
                        hi = plsc.bitcast(
                            jnp.bitwise_and(vp, jnp.int32(-65536)), jnp.float32)
                        lidx = jnp.full((16,), 2 * j, jnp.int32)
                        hidx = jnp.full((16,), 2 * j + 1, jnp.int32)
                        plsc.addupdate_scatter(acc_v, [lidx, d16], lo * w16)
                        plsc.addupdate_scatter(acc_v, [hidx, d16], hi * w16)
            return 0
        lax.fori_loop(0, nch // 2, outer, 0)

        pltpu.sync_copy(acc_v, out_hbm.at[pl.ds(2 * p0, F)])

    return spmm_kernel


def _dot(a, b):
    return lax.dot_general(a, b, (((1,), (0,)), ((), ())),
                           precision=lax.Precision.HIGHEST,
                           preferred_element_type=jnp.float32)


def _dotT(aT, b):
    return lax.dot_general(aT, b, (((0,), (0,)), ((), ())),
                           precision=lax.Precision.HIGHEST,
                           preferred_element_type=jnp.float32)


def _sigmoid(t):
    return 1.0 / (1.0 + jnp.exp(-t))


def _pack_pairs(a):
    hw = a.shape[1] // 2
    lo = lax.bitcast_convert_type(a[:, :hw].astype(jnp.bfloat16), jnp.uint16)
    hi = lax.bitcast_convert_type(a[:, hw:].astype(jnp.bfloat16), jnp.uint16)
    return (lo.astype(jnp.int32) | (hi.astype(jnp.int32) << 16))


@functools.lru_cache(maxsize=None)
def _make_tc_pre(N, D, NW, BN):
    grid = (N // BN,)

    def body(degT, x, h, wxz, whz, wxr, whr, wxh, bz, br, bh,
             dis_o, gz_o, gr_o, gxh_o, xs_o, hs_o):
        deg = jnp.sum(degT[...], axis=1)
        dis = jnp.where(deg > 0, lax.rsqrt(jnp.where(deg > 0, deg, 1.0)), 0.0)
        d = dis[:, None]
        dis_o[...] = d
        xx = x[...]
        hh = h[...]
        xs_o[...] = _pack_pairs(d * xx)
        hs_o[...] = _pack_pairs(d * hh)
        gz_o[...] = _dot(xx, wxz[...]) + _dot(hh, whz[...]) + bz[...]
        gr_o[...] = _dot(xx, wxr[...]) + _dot(hh, whr[...]) + br[...]
        gxh_o[...] = _dot(xx, wxh[...]) + bh[...]

    row_blk = pl.BlockSpec((BN, D), lambda i: (i, 0))
    pk_blk = pl.BlockSpec((BN, D // 2), lambda i: (i, 0))
    w_blk = pl.BlockSpec((D, D), lambda i: (0, 0))
    b_blk = pl.BlockSpec((1, D), lambda i: (0, 0))
    return pl.pallas_call(
        body, grid=grid,
        in_specs=[pl.BlockSpec((BN, NW), lambda i: (i, 0)), row_blk, row_blk,
                  w_blk, w_blk, w_blk, w_blk, w_blk, b_blk, b_blk, b_blk],
        out_specs=[pl.BlockSpec((BN, 1), lambda i: (i, 0)),
                   row_blk, row_blk, row_blk, pk_blk, pk_blk],
        out_shape=[jax.ShapeDtypeStruct((N, 1), jnp.float32),
                   jax.ShapeDtypeStruct((N, D), jnp.float32),
                   jax.ShapeDtypeStruct((N, D), jnp.float32),
                   jax.ShapeDtypeStruct((N, D), jnp.float32),
                   jax.ShapeDtypeStruct((N, D // 2), jnp.int32),
                   jax.ShapeDtypeStruct((N, D // 2), jnp.int32)],
    )


@functools.lru_cache(maxsize=None)
def _make_tc_mid(N, D, BN):
    grid = (N // BN,)

    def body(gz, gr, gxh, txt, tht, dis, h, wxz1, whz1, wxr1, whr1, whh0,
             z_o, hrs_o, gh_o):
        d = dis[...]
        tx = txt[...]
        th = tht[...]
        z = _sigmoid(gz[...] - d * (_dotT(tx, wxz1[...]) + _dotT(th, whz1[...])))
        r = _sigmoid(gr[...] - d * (_dotT(tx, wxr1[...]) + _dotT(th, whr1[...])))
        hr = h[...] * r
        z_o[...] = z
        hrs_o[...] = _pack_pairs(d * hr)
        gh_o[...] = gxh[...] + _dot(hr, whh0[...])

    row_blk = pl.BlockSpec((BN, D), lambda i: (i, 0))
    pk_blk = pl.BlockSpec((BN, D // 2), lambda i: (i, 0))
    t_blk = pl.BlockSpec((D, BN), lambda i: (0, i))
    w_blk = pl.BlockSpec((D, D), lambda i: (0, 0))
    return pl.pallas_call(
        body, grid=grid,
        in_specs=[row_blk, row_blk, row_blk, t_blk, t_blk,
                  pl.BlockSpec((BN, 1), lambda i: (i, 0)), row_blk,
                  w_blk, w_blk, w_blk, w_blk, w_blk],
        out_specs=[row_blk, pk_blk, row_blk],
        out_shape=[jax.ShapeDtypeStruct((N, D), jnp.float32),
                   jax.ShapeDtypeStruct((N, D // 2), jnp.int32),
                   jax.ShapeDtypeStruct((N, D), jnp.float32)],
    )


@functools.lru_cache(maxsize=None)
def _make_tc_fin(N, D, BN):
    grid = (N // BN,)

    def body(gh, txt, thrt, dis, z, h, wxh1, whh1, wlin, blin, out_o, h_o):
        d = dis[...]
        ht = jnp.tanh(gh[...] - d * (_dotT(txt[...], wxh1[...]) +
                                     _dotT(thrt[...], whh1[...])))
        zz = z[...]
        hv = zz * h[...] + (1.0 - zz) * ht
        h_o[...] = hv
        v = _dot(jnp.maximum(hv, 0.0), wlin[...]) + blin[...]
        out_o[...] = jnp.maximum(v, 0.0) + jnp.log1p(jnp.exp(-jnp.abs(v)))

    row_blk = pl.BlockSpec((BN, D), lambda i: (i, 0))
    t_blk = pl.BlockSpec((D, BN), lambda i: (0, i))
    w_blk = pl.BlockSpec((D, D), lambda i: (0, 0))
    return pl.pallas_call(
        body, grid=grid,
        in_specs=[row_blk, t_blk, t_blk,
                  pl.BlockSpec((BN, 1), lambda i: (i, 0)), row_blk, row_blk,
                  w_blk, w_blk, pl.BlockSpec((D, 1), lambda i: (0, 0)),
                  pl.BlockSpec((1, 1), lambda i: (0, 0))],
        out_specs=[pl.BlockSpec((BN, 1), lambda i: (i, 0)), row_blk],
        out_shape=[jax.ShapeDtypeStruct((N, 1), jnp.float32),
                   jax.ShapeDtypeStruct((N, D), jnp.float32)],
    )


def kernel(x, edge_index, edge_weight, h,
           W_xz, b_xz, W_hz, b_hz, W_xr, b_xr, W_hr, b_hr,
           W_xh, b_xh, W_hh, b_hh, W_lin, b_lin):
    N, D = x.shape
    E = edge_index.shape[1]
    info = plsc.get_sparse_core_info()
    NW = info.num_cores * info.num_subcores
    BN = 2048
    CH = 8000
    NP = -(-N // BN) * BN

    assert N <= 16384

    src = edge_index[0]
    dst = edge_index[1]
    xp = jnp.pad(x, ((0, NP - N), (0, 0)))
    hp = jnp.pad(h, ((0, NP - N), (0, 0)))

    deg_parts, pk = _make_sc_deg(NP, E)(src, dst, edge_weight)

    bz = (b_xz + b_hz).reshape(1, D)
    br = (b_xr + b_hr).reshape(1, D)
    bh = (b_xh + b_hh).reshape(1, D)
    dis, Gz, Gr, Gxh, xs_pk, hs_pk = _make_tc_pre(NP, D, NW, BN)(
        deg_parts.T, xp, hp, W_xz[0], W_hz[0], W_xr[0], W_hr[0], W_xh[0],
        bz, br, bh)

    DP = D // 2
    perm = jnp.array([p + b * DP for p in range(DP) for b in (0, 1)],
                     dtype=jnp.int32)

    spmm = _make_sc_spmm(NP, E, D, CH)
    TxT = spmm(pk, edge_weight, xs_pk.T)
    ThT = spmm(pk, edge_weight, hs_pk.T)

    Z, hrs_pk, Gh = _make_tc_mid(NP, D, BN)(
        Gz, Gr, Gxh, TxT, ThT, dis, hp,
        W_xz[1][perm], W_hz[1][perm], W_xr[1][perm], W_hr[1][perm], W_hh[0])

    ThrT = spmm(pk, edge_weight, hrs_pk.T)

    out, H = _make_tc_fin(NP, D, BN)(
        Gh, TxT, ThrT, dis, Z, hp, W_xh[1][perm], W_hh[1][perm], W_lin,
        b_lin.reshape(1, 1))
    return (out[:N], H[:N])


# speedup vs baseline: 1.1396x; 1.0042x over previous
import functools

import jax
import jax.numpy as jnp
from jax import lax
from jax.experimental import pallas as pl
from jax.experimental.pallas import tpu as pltpu
from jax.experimental.pallas import tpu_sc as plsc

_SC_PARAMS = None


def _sc_mesh():
    info = plsc.get_sparse_core_info()
    nc, ns = info.num_cores, info.num_subcores
    mesh = plsc.VectorSubcoreMesh(core_axis_name="c", subcore_axis_name="s")
    return mesh, nc, ns


def _sc_compiler_params():
    return pltpu.CompilerParams(needs_layout_passes=False)


@functools.lru_cache(maxsize=None)
def _make_sc_deg(N, E):
    mesh, nc, ns = _sc_mesh()
    nw = nc * ns
    assert E % (nw * 16) == 0
    ep = E // nw

    @functools.partial(
        pl.kernel, mesh=mesh,
        compiler_params=_sc_compiler_params(),
        out_type=(jax.ShapeDtypeStruct((nw, N), jnp.float32),
                  jax.ShapeDtypeStruct((E,), jnp.int32)),
        scratch_types=[
            pltpu.VMEM((ep,), jnp.int32),
            pltpu.VMEM((ep,), jnp.int32),
            pltpu.VMEM((ep,), jnp.float32),
            pltpu.VMEM((N,), jnp.float32),
        ],
    )
    def deg_kernel(src_hbm, dst_hbm, ew_hbm, out_hbm, pk_hbm,
                   src_v, dst_v, ew_v, acc_v):
        wid = lax.axis_index("s") * nc + lax.axis_index("c")
        base = wid * ep

        @plsc.parallel_loop(0, N // 16, unroll=8)
        def _zero(i):
            acc_v[pl.ds(i * 16, 16)] = jnp.zeros((16,), jnp.float32)

        pltpu.sync_copy(src_hbm.at[pl.ds(base, ep)], src_v)
        pltpu.sync_copy(dst_hbm.at[pl.ds(base, ep)], dst_v)
        pltpu.sync_copy(ew_hbm.at[pl.ds(base, ep)], ew_v)

        @plsc.parallel_loop(0, ep // 16, unroll=8)
        def _body(g):
            sl = pl.ds(g * 16, 16)
            idx = src_v[sl]
            w = ew_v[sl]
            plsc.addupdate_scatter(acc_v, [idx], w)
            dst_v[sl] = idx + lax.shift_left(dst_v[sl], 14)

        pltpu.sync_copy(acc_v, out_hbm.at[wid])
        pltpu.sync_copy(dst_v, pk_hbm.at[pl.ds(base, ep)])

    return deg_kernel


@functools.lru_cache(maxsize=None)
def _make_sc_spmm(N, E, D, CH):
    mesh, nc, ns = _sc_mesh()
    nw = nc * ns
    DP = D // 2
    assert DP % nw == 0 and E % CH == 0 and CH % 16 == 0
    FP = DP // nw
    F = 2 * FP
    nch = E // CH
    assert nch % 2 == 0

    @functools.partial(
        pl.kernel, mesh=mesh,
        compiler_params=_sc_compiler_params(),
        out_type=jax.ShapeDtypeStruct((D, N), jnp.float32),
        scratch_types=[
            pltpu.VMEM((FP, N), jnp.int32),
            pltpu.VMEM((F, N), jnp.float32),
            pltpu.VMEM((CH,), jnp.int32),
            pltpu.VMEM((CH,), jnp.int32),
            pltpu.VMEM((CH,), jnp.float32),
            pltpu.VMEM((CH,), jnp.float32),
            pltpu.SemaphoreType.DMA,
            pltpu.SemaphoreType.DMA,
        ],
    )
    def spmm_kernel(pk_hbm, ew_hbm, featP_hbm, out_hbm,
                    feat_v, acc_v, pk_v0, pk_v1, ew_v0, ew_v1, sem0, sem1):
        wid = lax.axis_index("s") * nc + lax.axis_index("c")
        p0 = wid * FP
        sems = (sem0, sem1)
        pk_b = (pk_v0, pk_v1)
        ew_b = (ew_v0, ew_v1)

        def start(c, b):
            base = c * CH
            pltpu.async_copy(pk_hbm.at[pl.ds(base, CH)], pk_b[b], sems[b])
            pltpu.async_copy(ew_hbm.at[pl.ds(base, CH)], ew_b[b], sems[b])

        def wait(b):
            pltpu.make_async_copy(pk_hbm.at[pl.ds(0, CH)], pk_b[b], sems[b]).wait()
            pltpu.make_async_copy(ew_hbm.at[pl.ds(0, CH)], ew_b[b], sems[b]).wait()

        start(0, 0)
        pltpu.sync_copy(featP_hbm.at[pl.ds(p0, FP)], feat_v)

        @plsc.parallel_loop(0, N // 16, unroll=8)
        def _zero(i):
            z = jnp.zeros((16,), jnp.float32)
            for f in range(F):
                acc_v[f, pl.ds(i * 16, 16)] = z

        def outer(i, _):
            for b in range(2):
                c = i * 2 + b

                @pl.when(c + 1 < nch)
                def _():
                    start(c + 1, 1 - b)

                wait(b)

                @plsc.parallel_loop(0, CH // 16, unroll=20)
                def _body(g):
                    pk16 = pk_b[b][pl.ds(g * 16, 16)]
                    w16 = ew_b[b][pl.ds(g * 16, 16)]
                    s16 = jnp.bitwise_and(pk16, 16383)
                    d16 = lax.shift_right_logical(pk16, 14)
                    for j in range(FP):
                        jidx = jnp.full((16,), j, jnp.int32)
                        vp = plsc.load_gather(feat_v, [jidx, s16])
                        lo = plsc.bitcast(lax.shift_left(vp, 16), jnp.float32)
                        hi = plsc.bitcast(
                            jnp.bitwise_and(vp, jnp.int32(-65536)), jnp.float32)
                        lidx = jnp.full((16,), 2 * j, jnp.int32)
                        hidx = jnp.full((16,), 2 * j + 1, jnp.int32)
                        plsc.addupdate_scatter(acc_v, [lidx, d16], lo * w16)
                        plsc.addupdate_scatter(acc_v, [hidx, d16], hi * w16)
            return 0
        lax.fori_loop(0, nch // 2, outer, 0)

        pltpu.sync_copy(acc_v, out_hbm.at[pl.ds(2 * p0, F)])

    return spmm_kernel


def _dot(a, b):
    return lax.dot_general(a, b, (((1,), (0,)), ((), ())),
                           precision=lax.Precision.HIGHEST,
                           preferred_element_type=jnp.float32)


def _dotT(aT, b):
    return lax.dot_general(aT, b, (((0,), (0,)), ((), ())),
                           precision=lax.Precision.HIGHEST,
                           preferred_element_type=jnp.float32)


def _sigmoid(t):
    return 1.0 / (1.0 + jnp.exp(-t))


def _pack_pairs(a):
    hw = a.shape[1] // 2
    lo = lax.bitcast_convert_type(a[:, :hw].astype(jnp.bfloat16), jnp.uint16)
    hi = lax.bitcast_convert_type(a[:, hw:].astype(jnp.bfloat16), jnp.uint16)
    return (lo.astype(jnp.int32) | (hi.astype(jnp.int32) << 16))


@functools.lru_cache(maxsize=None)
def _make_tc_pre(N, D, NW, BN):
    grid = (N // BN,)

    def body(degT, x, h, wxz, whz, wxr, whr, bz, br,
             dis_o, gz_o, gr_o, xs_o, hs_o):
        deg = jnp.sum(degT[...], axis=1)
        dis = jnp.where(deg > 0, lax.rsqrt(jnp.where(deg > 0, deg, 1.0)), 0.0)
        d = dis[:, None]
        dis_o[...] = d
        xx = x[...]
        hh = h[...]
        xs_o[...] = _pack_pairs(d * xx)
        hs_o[...] = _pack_pairs(d * hh)
        gz_o[...] = _dot(xx, wxz[...]) + _dot(hh, whz[...]) + bz[...]
        gr_o[...] = _dot(xx, wxr[...]) + _dot(hh, whr[...]) + br[...]

    row_blk = pl.BlockSpec((BN, D), lambda i: (i, 0))
    pk_blk = pl.BlockSpec((BN, D // 2), lambda i: (i, 0))
    w_blk = pl.BlockSpec((D, D), lambda i: (0, 0))
    b_blk = pl.BlockSpec((1, D), lambda i: (0, 0))
    return pl.pallas_call(
        body, grid=grid,
        in_specs=[pl.BlockSpec((BN, NW), lambda i: (i, 0)), row_blk, row_blk,
                  w_blk, w_blk, w_blk, w_blk, b_blk, b_blk],
        out_specs=[pl.BlockSpec((BN, 1), lambda i: (i, 0)),
                   row_blk, row_blk, pk_blk, pk_blk],
        out_shape=[jax.ShapeDtypeStruct((N, 1), jnp.float32),
                   jax.ShapeDtypeStruct((N, D), jnp.float32),
                   jax.ShapeDtypeStruct((N, D), jnp.float32),
                   jax.ShapeDtypeStruct((N, D // 2), jnp.int32),
                   jax.ShapeDtypeStruct((N, D // 2), jnp.int32)],
    )


@functools.lru_cache(maxsize=None)
def _make_tc_mid(N, D, BN):
    grid = (N // BN,)

    def body(gz, gr, x, txt, tht, dis, h, wxz1, whz1, wxr1, whr1, wxh0, whh0,
             bh, z_o, hrs_o, gh_o):
        d = dis[...]
        tx = txt[...]
        th = tht[...]
        z = _sigmoid(gz[...] - d * (_dotT(tx, wxz1[...]) + _dotT(th, whz1[...])))
        r = _sigmoid(gr[...] - d * (_dotT(tx, wxr1[...]) + _dotT(th, whr1[...])))
        hr = h[...] * r
        z_o[...] = z
        hrs_o[...] = _pack_pairs(d * hr)
        gh_o[...] = _dot(x[...], wxh0[...]) + bh[...] + _dot(hr, whh0[...])

    row_blk = pl.BlockSpec((BN, D), lambda i: (i, 0))
    pk_blk = pl.BlockSpec((BN, D // 2), lambda i: (i, 0))
    t_blk = pl.BlockSpec((D, BN), lambda i: (0, i))
    w_blk = pl.BlockSpec((D, D), lambda i: (0, 0))
    b_blk = pl.BlockSpec((1, D), lambda i: (0, 0))
    return pl.pallas_call(
        body, grid=grid,
        in_specs=[row_blk, row_blk, row_blk, t_blk, t_blk,
                  pl.BlockSpec((BN, 1), lambda i: (i, 0)), row_blk,
                  w_blk, w_blk, w_blk, w_blk, w_blk, w_blk, b_blk],
        out_specs=[row_blk, pk_blk, row_blk],
        out_shape=[jax.ShapeDtypeStruct((N, D), jnp.float32),
                   jax.ShapeDtypeStruct((N, D // 2), jnp.int32),
                   jax.ShapeDtypeStruct((N, D), jnp.float32)],
    )


@functools.lru_cache(maxsize=None)
def _make_tc_fin(N, D, BN):
    grid = (N // BN,)

    def body(gh, txt, thrt, dis, z, h, wxh1, whh1, wlin, blin, out_o, h_o):
        d = dis[...]
        ht = jnp.tanh(gh[...] - d * (_dotT(txt[...], wxh1[...]) +
                                     _dotT(thrt[...], whh1[...])))
        zz = z[...]
        hv = zz * h[...] + (1.0 - zz) * ht
        h_o[...] = hv
        v = _dot(jnp.maximum(hv, 0.0), wlin[...]) + blin[...]
        out_o[...] = jnp.maximum(v, 0.0) + jnp.log1p(jnp.exp(-jnp.abs(v)))

    row_blk = pl.BlockSpec((BN, D), lambda i: (i, 0))
    t_blk = pl.BlockSpec((D, BN), lambda i: (0, i))
    w_blk = pl.BlockSpec((D, D), lambda i: (0, 0))
    return pl.pallas_call(
        body, grid=grid,
        in_specs=[row_blk, t_blk, t_blk,
                  pl.BlockSpec((BN, 1), lambda i: (i, 0)), row_blk, row_blk,
                  w_blk, w_blk, pl.BlockSpec((D, 1), lambda i: (0, 0)),
                  pl.BlockSpec((1, 1), lambda i: (0, 0))],
        out_specs=[pl.BlockSpec((BN, 1), lambda i: (i, 0)), row_blk],
        out_shape=[jax.ShapeDtypeStruct((N, 1), jnp.float32),
                   jax.ShapeDtypeStruct((N, D), jnp.float32)],
    )


def kernel(x, edge_index, edge_weight, h,
           W_xz, b_xz, W_hz, b_hz, W_xr, b_xr, W_hr, b_hr,
           W_xh, b_xh, W_hh, b_hh, W_lin, b_lin):
    N, D = x.shape
    E = edge_index.shape[1]
    info = plsc.get_sparse_core_info()
    NW = info.num_cores * info.num_subcores
    BN = 2048
    CH = 8000
    NP = -(-N // BN) * BN

    assert N <= 16384

    src = edge_index[0]
    dst = edge_index[1]
    xp = jnp.pad(x, ((0, NP - N), (0, 0)))
    hp = jnp.pad(h, ((0, NP - N), (0, 0)))

    deg_parts, pk = _make_sc_deg(NP, E)(src, dst, edge_weight)

    bz = (b_xz + b_hz).reshape(1, D)
    br = (b_xr + b_hr).reshape(1, D)
    bh = (b_xh + b_hh).reshape(1, D)
    dis, Gz, Gr, xs_pk, hs_pk = _make_tc_pre(NP, D, NW, BN)(
        deg_parts.T, xp, hp, W_xz[0], W_hz[0], W_xr[0], W_hr[0], bz, br)

    DP = D // 2
    perm = jnp.array([p + b * DP for p in range(DP) for b in (0, 1)],
                     dtype=jnp.int32)

    spmm = _make_sc_spmm(NP, E, D, CH)
    TxT = spmm(pk, edge_weight, xs_pk.T)
    ThT = spmm(pk, edge_weight, hs_pk.T)

    Z, hrs_pk, Gh = _make_tc_mid(NP, D, BN)(
        Gz, Gr, xp, TxT, ThT, dis, hp,
        W_xz[1][perm], W_hz[1][perm], W_xr[1][perm], W_hr[1][perm],
        W_xh[0], W_hh[0], bh)

    ThrT = spmm(pk, edge_weight, hrs_pk.T)

    out, H = _make_tc_fin(NP, D, BN)(
        Gh, TxT, ThrT, dis, Z, hp, W_xh[1][perm], W_hh[1][perm], W_lin,
        b_lin.reshape(1, 1))
    return (out[:N], H[:N])
